# baseline (speedup 1.0000x reference)
import functools

import jax
import jax.numpy as jnp
from jax import lax
from jax.experimental import pallas as pl
from jax.experimental.pallas import tpu as pltpu
from jax.experimental.pallas import tpu_sc as plsc

NC = 2
NS = 16
L = 16
BINS = 256
SLAB = 32


def _sc_hist_resize(x3, ch, in_h, in_w):
    cpc = ch // NC
    nslab = in_h // SLAB
    mesh = plsc.VectorSubcoreMesh(
        core_axis_name="c", subcore_axis_name="s", num_cores=NC, num_subcores=NS
    )

    @functools.partial(
        pl.kernel,
        out_type=jax.ShapeDtypeStruct((ch, in_h, in_w), jnp.float32),
        mesh=mesh,
        compiler_params=pltpu.CompilerParams(needs_layout_passes=False),
        scratch_types=[
            pltpu.VMEM((SLAB, in_w), jnp.float32),
            pltpu.VMEM((SLAB, in_w), jnp.float32),
            pltpu.VMEM((L * BINS,), jnp.float32),
            pltpu.VMEM((BINS,), jnp.float32),
            pltpu.VMEM((in_h,), jnp.float32),
            pltpu.VMEM((SLAB, in_w), jnp.float32),
            pltpu.VMEM((SLAB, in_w), jnp.float32),
            pltpu.SemaphoreType.DMA,
            pltpu.SemaphoreType.DMA,
            pltpu.SemaphoreType.DMA,
            pltpu.SemaphoreType.DMA,
        ],
    )
    def hist_kernel(x_hbm, out_hbm, buf_a, buf_b, sub, hist, vals, ob_a, ob_b,
                    sem_a, sem_b, osem_a, osem_b):
        core = lax.axis_index("c")
        sid = lax.axis_index("s")
        mych = core * cpc + sid
        bufs = (buf_a, buf_b)
        sems = (sem_a, sem_b)
        obufs = (ob_a, ob_b)
        osems = (osem_a, osem_b)
        lanebase = lax.broadcasted_iota(jnp.int32, (L,), 0) * BINS
        ones = jnp.ones((L,), jnp.float32)
        zeros = jnp.zeros((L,), jnp.float32)

        @pl.when(sid < cpc)
        def _active():
            @plsc.parallel_loop(0, L * BINS, step=L, unroll=4)
            def zero_body(i):
                sub[pl.ds(i, L)] = zeros

            def issue(s, par):
                return pltpu.async_copy(
                    x_hbm.at[mych, pl.ds(s * SLAB, SLAB), :],
                    bufs[par], sems[par]
                )

            def wait(par):
                pltpu.make_async_copy(
                    x_hbm.at[0, pl.ds(0, SLAB), :], bufs[par], sems[par]
                ).wait()

            def process(buf):
                @plsc.parallel_loop(0, in_w, step=L)
                def h_body(i):
                    for r in range(SLAB):
                        v = buf[r, pl.ds(i, L)]
                        idx = (v * 256.0).astype(jnp.int32)
                        plsc.addupdate_scatter(sub, [lanebase + idx], ones)

            issue(0, 0)

            def pair_body(k, carry):
                issue(2 * k + 1, 1)
                wait(0)
                process(buf_a)

                @pl.when(k < nslab // 2 - 1)
                def _():
                    issue(2 * k + 2, 0)

                wait(1)
                process(buf_b)
                return carry

            lax.fori_loop(0, nslab // 2, pair_body, None)

            @plsc.parallel_loop(0, BINS, step=L, unroll=2)
            def r_body(j):
                vs = [sub[pl.ds(r * BINS + j, L)] for r in range(L)]
                while len(vs) > 1:
                    vs = [a + b for a, b in zip(vs[::2], vs[1::2])]
                hist[pl.ds(j, L)] = vs[0]

            def v_body(g, carry):
                rv = (g * L + lax.broadcasted_iota(jnp.int32, (L,), 0)
                      ).astype(jnp.float32)
                ys = jnp.maximum(rv * 0.5 - 0.25, 0.0)
                y0 = ys.astype(jnp.int32)
                wy = ys - y0.astype(jnp.float32)
                y1 = jnp.minimum(y0 + 1, BINS - 1)
                h0 = plsc.load_gather(hist, [y0])
                h1 = plsc.load_gather(hist, [y1])
                vals[pl.ds(g * L, L)] = h0 * (1.0 - wy) + wy * h1
                return carry

            lax.fori_loop(0, in_h // L, v_body, None)

            def fill_slab(s, ob):
                def row_body(j, carry):
                    bv = plsc.load_gather(
                        vals, [jnp.full((L,), 0, jnp.int32) + (s * SLAB + j)])
                    for k in range(in_w // L):
                        ob[j, pl.ds(k * L, L)] = bv
                    return carry

                lax.fori_loop(0, SLAB, row_body, None)

            def oissue(s, par):
                return pltpu.async_copy(
                    obufs[par], out_hbm.at[mych, pl.ds(s * SLAB, SLAB), :],
                    osems[par]
                )

            def owait(par):
                pltpu.make_async_copy(
                    obufs[par], out_hbm.at[0, pl.ds(0, SLAB), :], osems[par]
                ).wait()

            def out_body(k, carry):
                @pl.when(k > 0)
                def _():
                    owait(0)

                fill_slab(2 * k, ob_a)
                oissue(2 * k, 0)

                @pl.when(k > 0)
                def _():
                    owait(1)

                fill_slab(2 * k + 1, ob_b)
                oissue(2 * k + 1, 1)
                return carry

            lax.fori_loop(0, nslab // 2, out_body, None)
            owait(0)
            owait(1)

    return hist_kernel(x3)


def kernel(x):
    b, c, h, w = x.shape
    ch = b * c
    x3 = x.reshape(ch, h, w)
    out = _sc_hist_resize(x3, ch, h, w)
    return out.reshape(b, c, h, w)


# speedup vs baseline: 1.1264x; 1.1264x over previous
import functools

import jax
import jax.numpy as jnp
from jax import lax
from jax.experimental import pallas as pl
from jax.experimental.pallas import tpu as pltpu
from jax.experimental.pallas import tpu_sc as plsc

NC = 2
NS = 16
L = 16
NW = NC * NS
BINS = 256


def _sc_partial_hists(x3, ch, in_h, in_w):
    rows_pt = in_h // NS
    cpc = ch // NC
    mesh = plsc.VectorSubcoreMesh(
        core_axis_name="c", subcore_axis_name="s", num_cores=NC, num_subcores=NS
    )

    @functools.partial(
        pl.kernel,
        out_type=jax.ShapeDtypeStruct((NC, NS, cpc * BINS), jnp.float32),
        mesh=mesh,
        compiler_params=pltpu.CompilerParams(needs_layout_passes=False),
        scratch_types=[
            pltpu.VMEM((rows_pt, in_w), jnp.float32),
            pltpu.VMEM((rows_pt, in_w), jnp.float32),
            pltpu.VMEM((L * BINS,), jnp.float32),
            pltpu.VMEM((cpc * BINS,), jnp.float32),
            pltpu.SemaphoreType.DMA,
            pltpu.SemaphoreType.DMA,
        ],
    )
    def hist_kernel(x_hbm, out_hbm, buf_a, buf_b, sub, red, sem_a, sem_b):
        core = lax.axis_index("c")
        sid = lax.axis_index("s")
        ch0 = core * cpc
        lanebase = lax.broadcasted_iota(jnp.int32, (L,), 0) * BINS
        ones = jnp.ones((L,), jnp.float32)
        zeros = jnp.zeros((L,), jnp.float32)
        bufs = (buf_a, buf_b)
        sems = (sem_a, sem_b)

        @plsc.parallel_loop(0, L * BINS, step=L, unroll=4)
        def zero_body(i):
            sub[pl.ds(i, L)] = zeros

        def issue(c, par):
            return pltpu.async_copy(
                x_hbm.at[ch0 + c, pl.ds(sid * rows_pt, rows_pt), :],
                bufs[par], sems[par]
            )

        def wait(par):
            pltpu.make_async_copy(
                x_hbm.at[0, pl.ds(0, rows_pt), :], bufs[par], sems[par]
            ).wait()

        def process(c, buf):
            @plsc.parallel_loop(0, in_w, step=L)
            def h_body(i):
                for r in range(rows_pt):
                    v = buf[r, pl.ds(i, L)]
                    idx = (v * 256.0).astype(jnp.int32)
                    plsc.addupdate_scatter(sub, [lanebase + idx], ones)

            @plsc.parallel_loop(0, BINS, step=L, unroll=2)
            def r_body(j):
                vs = []
                for r in range(L):
                    off = r * BINS + j
                    vs.append(sub[pl.ds(off, L)])
                    sub[pl.ds(off, L)] = zeros
                while len(vs) > 1:
                    vs = [a + b for a, b in zip(vs[::2], vs[1::2])]
                red[pl.ds(c * BINS + j, L)] = vs[0]

        issue(0, 0)

        def pair_body(k, carry):
            c0 = 2 * k
            issue(c0 + 1, 1)
            wait(0)
            process(c0, buf_a)

            @pl.when(k < cpc // 2 - 1)
            def _():
                issue(c0 + 2, 0)

            wait(1)
            process(c0 + 1, buf_b)
            return carry

        lax.fori_loop(0, cpc // 2, pair_body, None)
        pltpu.sync_copy(red, out_hbm.at[core, sid])

    return hist_kernel(x3)


def _tc_expand(partials, ch, out_h, out_w):
    cb = 4
    steps = ch // cb
    spc = steps // NC

    def body(p_ref, o_ref):
        yi = lax.broadcasted_iota(jnp.int32, (out_h, BINS), 0).astype(jnp.float32)
        ki = lax.broadcasted_iota(jnp.int32, (out_h, BINS), 1).astype(jnp.float32)
        ys = jnp.maximum(yi * (BINS / out_h) + (0.5 * BINS / out_h - 0.5), 0.0)
        y0 = jnp.floor(ys)
        wy = ys - y0
        y1 = jnp.minimum(y0 + 1.0, float(BINS - 1))
        stencil = (jnp.where(ki == y0, 1.0 - wy, 0.0)
                   + jnp.where(ki == y1, wy, 0.0))
        for k in range(cb):
            h_row = jnp.sum(p_ref[0, :, k * BINS:(k + 1) * BINS],
                            axis=0, keepdims=True)
            vals = jnp.sum(stencil * h_row, axis=1, keepdims=True)
            o_ref[k] = jnp.broadcast_to(vals, (out_h, out_w))

    return pl.pallas_call(
        body,
        grid=(steps,),
        in_specs=[pl.BlockSpec((1, NS, cb * BINS), lambda g: (g // spc, 0, g % spc))],
        out_specs=pl.BlockSpec((cb, out_h, out_w), lambda g: (g, 0, 0)),
        out_shape=jax.ShapeDtypeStruct((ch, out_h, out_w), jnp.float32),
    )(partials)


def kernel(x):
    b, c, h, w = x.shape
    ch = b * c
    x3 = x.reshape(ch, h, w)
    partials = _sc_partial_hists(x3, ch, h, w)
    out = _tc_expand(partials, ch, h, w)
    return out.reshape(b, c, h, w)


# speedup vs baseline: 1.1920x; 1.0582x over previous
import functools

import jax
import jax.numpy as jnp
from jax import lax
from jax.experimental import pallas as pl
from jax.experimental.pallas import tpu as pltpu
from jax.experimental.pallas import tpu_sc as plsc

NC = 2
NS = 16
L = 16
NW = NC * NS
BINS = 256


def _sc_partial_hists(x3, ch, in_h, in_w):
    rows_pt = in_h // NW
    mesh = plsc.VectorSubcoreMesh(
        core_axis_name="c", subcore_axis_name="s", num_cores=NC, num_subcores=NS
    )

    @functools.partial(
        pl.kernel,
        out_type=jax.ShapeDtypeStruct((NW, ch * BINS), jnp.float32),
        mesh=mesh,
        compiler_params=pltpu.CompilerParams(needs_layout_passes=False),
        scratch_types=[
            pltpu.VMEM((rows_pt, in_w), jnp.float32),
            pltpu.VMEM((rows_pt, in_w), jnp.float32),
            pltpu.VMEM((L * BINS,), jnp.float32),
            pltpu.VMEM((ch * BINS,), jnp.float32),
            pltpu.SemaphoreType.DMA,
            pltpu.SemaphoreType.DMA,
        ],
    )
    def hist_kernel(x_hbm, out_hbm, buf_a, buf_b, sub, red, sem_a, sem_b):
        wid = lax.axis_index("s") * NC + lax.axis_index("c")
        lanebase = lax.broadcasted_iota(jnp.int32, (L,), 0) * BINS
        ones = jnp.ones((L,), jnp.float32)
        zeros = jnp.zeros((L,), jnp.float32)
        bufs = (buf_a, buf_b)
        sems = (sem_a, sem_b)

        @plsc.parallel_loop(0, L * BINS, step=L, unroll=4)
        def zero_body(i):
            sub[pl.ds(i, L)] = zeros

        def issue(c, par):
            return pltpu.async_copy(
                x_hbm.at[c, pl.ds(wid * rows_pt, rows_pt), :],
                bufs[par], sems[par]
            )

        def wait(par):
            pltpu.make_async_copy(
                x_hbm.at[0, pl.ds(0, rows_pt), :], bufs[par], sems[par]
            ).wait()

        def process(c, buf):
            @plsc.parallel_loop(0, in_w, step=L, unroll=2)
            def h_body(i):
                for r in range(rows_pt):
                    v = buf[r, pl.ds(i, L)]
                    idx = (v * 256.0).astype(jnp.int32)
                    plsc.addupdate_scatter(sub, [lanebase + idx], ones)

            @plsc.parallel_loop(0, BINS, step=L, unroll=2)
            def r_body(j):
                vs = []
                for r in range(L):
                    off = r * BINS + j
                    vs.append(sub[pl.ds(off, L)])
                    sub[pl.ds(off, L)] = zeros
                while len(vs) > 1:
                    vs = [a + b for a, b in zip(vs[::2], vs[1::2])]
                red[pl.ds(c * BINS + j, L)] = vs[0]

        issue(0, 0)

        def pair_body(k, carry):
            c0 = 2 * k
            issue(c0 + 1, 1)
            wait(0)
            process(c0, buf_a)

            @pl.when(k < ch // 2 - 1)
            def _():
                issue(c0 + 2, 0)

            wait(1)
            process(c0 + 1, buf_b)
            return carry

        lax.fori_loop(0, ch // 2, pair_body, None)
        pltpu.sync_copy(red, out_hbm.at[wid])

    return hist_kernel(x3)


def _tc_expand(partials, ch, out_h, out_w):
    cb = 8
    steps = ch // cb

    def body(p_ref, o_ref):
        yi = lax.broadcasted_iota(jnp.int32, (out_h, BINS), 0).astype(jnp.float32)
        ki = lax.broadcasted_iota(jnp.int32, (out_h, BINS), 1).astype(jnp.float32)
        ys = jnp.maximum(yi * (BINS / out_h) + (0.5 * BINS / out_h - 0.5), 0.0)
        y0 = jnp.floor(ys)
        wy = ys - y0
        y1 = jnp.minimum(y0 + 1.0, float(BINS - 1))
        stencil = (jnp.where(ki == y0, 1.0 - wy, 0.0)
                   + jnp.where(ki == y1, wy, 0.0))
        for k in range(cb):
            h_row = jnp.sum(p_ref[:, k * BINS:(k + 1) * BINS],
                            axis=0, keepdims=True)
            vals = jnp.sum(stencil * h_row, axis=1, keepdims=True)
            o_ref[k] = jnp.broadcast_to(vals, (out_h, out_w))

    return pl.pallas_call(
        body,
        grid=(steps,),
        in_specs=[pl.BlockSpec((NW, cb * BINS), lambda g: (0, g))],
        out_specs=pl.BlockSpec((cb, out_h, out_w), lambda g: (g, 0, 0)),
        out_shape=jax.ShapeDtypeStruct((ch, out_h, out_w), jnp.float32),
    )(partials)


def kernel(x):
    b, c, h, w = x.shape
    ch = b * c
    x3 = x.reshape(ch, h, w)
    partials = _sc_partial_hists(x3, ch, h, w)
    out = _tc_expand(partials, ch, h, w)
    return out.reshape(b, c, h, w)
